# baseline (speedup 1.0000x reference)
import functools

import jax
import jax.numpy as jnp
from jax import lax
from jax.experimental import pallas as pl
from jax.experimental.pallas import tpu as pltpu
from jax.experimental.pallas import tpu_sc as plsc

NUM_EXPERTS = 64
TOP_K = 8
N_GROUP = 8
TOPK_GROUP = 4
EPG = NUM_EXPERTS // N_GROUP
HIDDEN = 2048
TOKENS = 8192
SCALING = 2.5

NC, NS, L = 2, 16, 16
NW = NC * NS
TPW = TOKENS // NW
NB = TPW // L

TB = 512


def _score_body(h_ref, wt_ref, b_ref, o_ref):
    logits = jnp.dot(h_ref[...], wt_ref[...], preferred_element_type=jnp.float32)
    o_ref[...] = jax.nn.sigmoid(logits) + b_ref[...]


def _scores(hidden_states, w_t, bias2d):
    return pl.pallas_call(
        _score_body,
        grid=(TOKENS // TB,),
        in_specs=[
            pl.BlockSpec((TB, HIDDEN), lambda i: (i, 0)),
            pl.BlockSpec((HIDDEN, NUM_EXPERTS), lambda i: (0, 0)),
            pl.BlockSpec((1, NUM_EXPERTS), lambda i: (0, 0)),
        ],
        out_specs=pl.BlockSpec((TB, NUM_EXPERTS), lambda i: (i, 0)),
        out_shape=jax.ShapeDtypeStruct((TOKENS, NUM_EXPERTS), jnp.float32),
    )(hidden_states, w_t, bias2d)


_mesh = plsc.VectorSubcoreMesh(
    core_axis_name="c", subcore_axis_name="s", num_cores=NC, num_subcores=NS
)


@functools.partial(
    pl.kernel,
    out_type=(
        jax.ShapeDtypeStruct((TOKENS * TOP_K,), jnp.float32),
        jax.ShapeDtypeStruct((TOKENS * TOP_K,), jnp.int32),
    ),
    mesh=_mesh,
    compiler_params=pltpu.CompilerParams(needs_layout_passes=False),
    scratch_types=[
        pltpu.VMEM((TPW * NUM_EXPERTS,), jnp.float32),
        pltpu.VMEM((NUM_EXPERTS,), jnp.float32),
        pltpu.VMEM((NUM_EXPERTS * L,), jnp.float32),
        pltpu.VMEM((TPW * TOP_K,), jnp.float32),
        pltpu.VMEM((TPW * TOP_K,), jnp.int32),
    ],
)
def _route(corr_hbm, bias_hbm, rw_hbm, se_hbm, corr_v, bias_v, masked_v, rw_v, se_v):
    wid = lax.axis_index("s") * NC + lax.axis_index("c")
    base = wid * TPW
    pltpu.sync_copy(corr_hbm.at[pl.ds(base * NUM_EXPERTS, TPW * NUM_EXPERTS)], corr_v)
    pltpu.sync_copy(bias_hbm, bias_v)
    iota = lax.iota(jnp.int32, L)
    neg = jnp.full((L,), -jnp.inf, jnp.float32)
    zero_i = jnp.zeros((L,), jnp.int32)

    def batch(b, carry):
        tvec = b * L + iota
        tE = tvec * NUM_EXPERTS
        tK = tvec * TOP_K

        gs = []
        for g in range(N_GROUP):
            top1 = neg
            top2 = neg
            for j in range(EPG):
                e = g * EPG + j
                v = plsc.load_gather(corr_v, [tE + e])
                gt = v > top1
                top2 = jnp.where(gt, top1, jnp.maximum(top2, v))
                top1 = jnp.where(gt, v, top1)
            gs.append(top1 + top2)

        gmask = [jnp.zeros((L,), jnp.float32) for _ in range(N_GROUP)]
        for _ in range(TOPK_GROUP):
            best = neg
            bestg = zero_i
            for g in range(N_GROUP):
                gt = gs[g] > best
                best = jnp.where(gt, gs[g], best)
                bestg = jnp.where(gt, jnp.int32(g), bestg)
            for g in range(N_GROUP):
                sel = bestg == g
                gmask[g] = jnp.where(sel, 1.0, gmask[g])
                gs[g] = jnp.where(sel, neg, gs[g])

        for e in range(NUM_EXPERTS):
            v = plsc.load_gather(corr_v, [tE + e])
            masked_v[pl.ds(e * L, L)] = v * gmask[e // EPG]

        rws = []
        ses = []
        for _ in range(TOP_K):
            best = neg
            beste = zero_i
            for e in range(NUM_EXPERTS):
                v = masked_v[pl.ds(e * L, L)]
                gt = v > best
                best = jnp.where(gt, v, best)
                beste = jnp.where(gt, jnp.int32(e), beste)
            ses.append(beste)
            cv = plsc.load_gather(corr_v, [tE + beste])
            bv = plsc.load_gather(bias_v, [beste])
            rws.append(cv - bv)
            plsc.store_scatter(masked_v, [beste * L + iota], neg)

        tot = rws[0]
        for k in range(1, TOP_K):
            tot = tot + rws[k]
        scale = jnp.float32(SCALING) / (tot + jnp.float32(1e-20))
        for k in range(TOP_K):
            plsc.store_scatter(rw_v, [tK + k], rws[k] * scale)
            plsc.store_scatter(se_v, [tK + k], ses[k])
        return carry

    lax.fori_loop(0, NB, batch, 0)
    pltpu.sync_copy(rw_v, rw_hbm.at[pl.ds(base * TOP_K, TPW * TOP_K)])
    pltpu.sync_copy(se_v, se_hbm.at[pl.ds(base * TOP_K, TPW * TOP_K)])


def kernel(hidden_states, weight, e_score_correction_bias):
    w_t = weight.T
    bias2d = e_score_correction_bias[None, :]
    corrected = _scores(hidden_states, w_t, bias2d)
    rw_flat, se_flat = _route(corrected.reshape(-1), e_score_correction_bias)
    return (
        rw_flat.reshape(TOKENS, TOP_K),
        se_flat.reshape(TOKENS, TOP_K),
    )


# speedup vs baseline: 1.3939x; 1.3939x over previous
import functools

import jax
import jax.numpy as jnp
from jax import lax
from jax.experimental import pallas as pl
from jax.experimental.pallas import tpu as pltpu
from jax.experimental.pallas import tpu_sc as plsc

NUM_EXPERTS = 64
TOP_K = 8
N_GROUP = 8
TOPK_GROUP = 4
EPG = NUM_EXPERTS // N_GROUP
HIDDEN = 2048
TOKENS = 8192
SCALING = 2.5

NC, NS, L = 2, 16, 16
NW = NC * NS
TPW = TOKENS // NW
NB = TPW // L

TB = 1024


def _score_body(h_ref, wt_ref, b_ref, o_ref):
    logits = jnp.dot(h_ref[...], wt_ref[...], preferred_element_type=jnp.float32)
    o_ref[...] = jax.nn.sigmoid(logits) + b_ref[...]


def _scores(hidden_states, w_t, bias2d):
    return pl.pallas_call(
        _score_body,
        grid=(TOKENS // TB,),
        in_specs=[
            pl.BlockSpec((TB, HIDDEN), lambda i: (i, 0)),
            pl.BlockSpec((HIDDEN, NUM_EXPERTS), lambda i: (0, 0)),
            pl.BlockSpec((1, NUM_EXPERTS), lambda i: (0, 0)),
        ],
        out_specs=pl.BlockSpec((TB, NUM_EXPERTS), lambda i: (i, 0)),
        out_shape=jax.ShapeDtypeStruct((TOKENS, NUM_EXPERTS), jnp.float32),
    )(hidden_states, w_t, bias2d)


_mesh = plsc.VectorSubcoreMesh(
    core_axis_name="c", subcore_axis_name="s", num_cores=NC, num_subcores=NS
)


def _merge(av, ai, bv, bi):
    gt = bv > av
    return jnp.where(gt, bv, av), jnp.where(gt, bi, ai)


@functools.partial(
    pl.kernel,
    out_type=(
        jax.ShapeDtypeStruct((TOKENS * TOP_K,), jnp.float32),
        jax.ShapeDtypeStruct((TOKENS * TOP_K,), jnp.int32),
    ),
    mesh=_mesh,
    compiler_params=pltpu.CompilerParams(needs_layout_passes=False),
    scratch_types=[
        pltpu.VMEM((TPW * NUM_EXPERTS,), jnp.float32),
        pltpu.VMEM((NUM_EXPERTS,), jnp.float32),
        pltpu.VMEM((NUM_EXPERTS * L,), jnp.float32),
        pltpu.VMEM((TPW * TOP_K,), jnp.float32),
        pltpu.VMEM((TPW * TOP_K,), jnp.int32),
    ],
)
def _route(corr_hbm, bias_hbm, rw_hbm, se_hbm, corr_v, bias_v, work_v, rw_v, se_v):
    wid = lax.axis_index("s") * NC + lax.axis_index("c")
    base = wid * TPW
    pltpu.sync_copy(corr_hbm.at[pl.ds(base * NUM_EXPERTS, TPW * NUM_EXPERTS)], corr_v)
    pltpu.sync_copy(bias_hbm, bias_v)
    iota = lax.iota(jnp.int32, L)
    neg = jnp.full((L,), -jnp.inf, jnp.float32)
    zero_i = jnp.zeros((L,), jnp.int32)

    def batch(b, carry):
        tvec = b * L + iota
        tE = tvec * NUM_EXPERTS
        tK = tvec * TOP_K

        top1 = []
        idx1 = []
        gs = []
        for g in range(N_GROUP):
            t1 = neg
            t2 = neg
            i1 = zero_i
            for j in range(EPG):
                e = g * EPG + j
                v = plsc.load_gather(corr_v, [tE + e])
                work_v[pl.ds(e * L, L)] = v
                gt = v > t1
                t2 = jnp.where(gt, t1, jnp.maximum(t2, v))
                i1 = jnp.where(gt, jnp.int32(e), i1)
                t1 = jnp.where(gt, v, t1)
            top1.append(t1)
            idx1.append(i1)
            gs.append(t1 + t2)

        gmask = [jnp.zeros((L,), jnp.float32) for _ in range(N_GROUP)]
        for _ in range(TOPK_GROUP):
            m01 = _merge(gs[0], zero_i, gs[1], zero_i + 1)
            m23 = _merge(gs[2], zero_i + 2, gs[3], zero_i + 3)
            m45 = _merge(gs[4], zero_i + 4, gs[5], zero_i + 5)
            m67 = _merge(gs[6], zero_i + 6, gs[7], zero_i + 7)
            ma = _merge(*m01, *m23)
            mb = _merge(*m45, *m67)
            _, bestg = _merge(*ma, *mb)
            for g in range(N_GROUP):
                sel = bestg == g
                gmask[g] = jnp.where(sel, 1.0, gmask[g])
                gs[g] = jnp.where(sel, neg, gs[g])

        gmax = []
        gae = []
        for g in range(N_GROUP):
            z = plsc.load_gather(corr_v, [tE + g * EPG]) * jnp.float32(0.0)
            sel = gmask[g] > 0.0
            gmax.append(jnp.where(sel, top1[g], z))
            gae.append(jnp.where(sel, idx1[g], jnp.int32(g * EPG)))

        rws = []
        tot = None
        for k in range(TOP_K):
            m01 = _merge(gmax[0], gae[0], gmax[1], gae[1])
            m23 = _merge(gmax[2], gae[2], gmax[3], gae[3])
            m45 = _merge(gmax[4], gae[4], gmax[5], gae[5])
            m67 = _merge(gmax[6], gae[6], gmax[7], gae[7])
            ma = _merge(*m01, *m23)
            mb = _merge(*m45, *m67)
            _, beste = _merge(*ma, *mb)
            plsc.store_scatter(se_v, [tK + k], beste)
            cv = plsc.load_gather(corr_v, [tE + beste])
            bv = plsc.load_gather(bias_v, [beste])
            w = cv - bv
            rws.append(w)
            tot = w if tot is None else tot + w
            plsc.store_scatter(work_v, [beste * L + iota], neg)
            bestg = lax.shift_right_logical(beste, 3)
            msel = gmask[0]
            for g in range(1, N_GROUP):
                msel = jnp.where(bestg == g, gmask[g], msel)
            eb = lax.shift_left(bestg, 3)
            ebL = lax.shift_left(eb, 4) + iota
            gm = neg
            ga = zero_i
            for j in range(EPG):
                v = plsc.load_gather(work_v, [ebL + j * L]) * msel
                gt = v > gm
                gm = jnp.where(gt, v, gm)
                ga = jnp.where(gt, eb + j, ga)
            for g in range(N_GROUP):
                sel = bestg == g
                gmax[g] = jnp.where(sel, gm, gmax[g])
                gae[g] = jnp.where(sel, ga, gae[g])

        scale = jnp.float32(SCALING) / (tot + jnp.float32(1e-20))
        for k in range(TOP_K):
            plsc.store_scatter(rw_v, [tK + k], rws[k] * scale)
        return carry

    lax.fori_loop(0, NB, batch, 0)
    pltpu.sync_copy(rw_v, rw_hbm.at[pl.ds(base * TOP_K, TPW * TOP_K)])
    pltpu.sync_copy(se_v, se_hbm.at[pl.ds(base * TOP_K, TPW * TOP_K)])


def kernel(hidden_states, weight, e_score_correction_bias):
    w_t = weight.T
    bias2d = e_score_correction_bias[None, :]
    corrected = _scores(hidden_states, w_t, bias2d)
    rw_flat, se_flat = _route(corrected.reshape(-1), e_score_correction_bias)
    return (
        rw_flat.reshape(TOKENS, TOP_K),
        se_flat.reshape(TOKENS, TOP_K),
    )
